# baseline (speedup 1.0000x reference)
import functools
import math

import jax
import jax.numpy as jnp
from jax import lax
from jax.experimental import pallas as pl
from jax.experimental.pallas import tpu as pltpu
from jax.experimental.pallas import tpu_sc as plsc

EMB = 64
SCALE = math.sqrt(EMB)

_NC = 2
_NS = 16
_NW = _NC * _NS

_BLK = 256
_SEQ = 200
_S = 16384
_SPW = _S // _NW


def _tile_body(table_hbm, x_hbm, out_hbm, idx_v, rows_v, tbuf_v, isem, gsem,
               wsem):
  wid = lax.axis_index("s") * _NC + lax.axis_index("c")
  n_blocks = _SEQ * (_SPW // _BLK)

  iota16 = lax.iota(jnp.int32, 16)

  def xoff(k):
    t = k // 2
    sub = k % 2
    return t * _S + wid * _SPW + sub * _BLK

  def start_idx(k, b):
    pltpu.async_copy(x_hbm.at[pl.ds(xoff(k), _BLK)], idx_v.at[b], isem[b])

  def wait_idx(b):
    pltpu.make_async_copy(x_hbm.at[pl.ds(0, _BLK)], idx_v.at[b],
                          isem[b]).wait()

  def start_gather(b):
    pltpu.async_copy(table_hbm.at[idx_v.at[b]], rows_v.at[b], gsem[b])

  def wait_gather(b):
    pltpu.make_async_copy(table_hbm.at[idx_v.at[b]], rows_v.at[b],
                          gsem[b]).wait()

  def start_write(k, b):
    t = k // 2
    sub = k % 2
    j0 = wid * 4 + sub * 2
    pltpu.async_copy(tbuf_v.at[b], out_hbm.at[t, :, pl.ds(j0, 2), :],
                     wsem[b])

  def wait_write(b):
    pltpu.make_async_copy(tbuf_v.at[b], out_hbm.at[0, :, pl.ds(0, 2), :],
                          wsem[b]).wait()

  def transpose_scale(b):
    rows = rows_v.at[b]
    tb = tbuf_v.at[b]

    def qj_body(qj, carry):
      q = qj // 2
      j = qj % 2
      for e8 in range(8):
        col = jnp.full((16,), q * 8 + e8, jnp.int32)
        for c in range(8):
          row_ids = iota16 + (j * 128 + c * 16)
          vals = plsc.load_gather(rows, [row_ids, col])
          tb[q, j, pl.ds(e8 * 128 + c * 16, 16)] = vals * SCALE
      return carry

    lax.fori_loop(0, 16, qj_body, 0)

  def step(k, b, *, do_idx=True, do_gather=True, do_wwait=True):
    b1 = (b + 1) % 2
    wait_gather(b)
    if do_idx:
      start_idx(k + 2, b)
    if do_gather:
      wait_idx(b1)
      start_gather(b1)
    if do_wwait:
      wait_write(b)
    transpose_scale(b)
    start_write(k, b)

  start_idx(0, 0)
  start_idx(1, 1)
  wait_idx(0)
  start_gather(0)
  step(0, 0, do_wwait=False)
  step(1, 1, do_wwait=False)

  def pair(p, carry):
    step(2 * p, 0)
    step(2 * p + 1, 1)
    return carry

  lax.fori_loop(1, n_blocks // 2 - 1, pair, 0)

  step(n_blocks - 2, 0, do_idx=False)
  step(n_blocks - 1, 1, do_idx=False, do_gather=False)
  wait_write(0)
  wait_write(1)


@jax.jit
def _lookup(table, xt):
  mesh = plsc.VectorSubcoreMesh(core_axis_name="c", subcore_axis_name="s",
                                num_cores=_NC)
  return pl.kernel(
      _tile_body,
      out_type=jax.ShapeDtypeStruct((_SEQ, 8, _S // 128, 1024), jnp.float32),
      mesh=mesh,
      scratch_types=[
          pltpu.VMEM((2, _BLK), jnp.int32),
          pltpu.VMEM((2, _BLK, EMB), jnp.float32),
          pltpu.VMEM((2, 8, 2, 1024), jnp.float32),
          [pltpu.SemaphoreType.DMA] * 2,
          [pltpu.SemaphoreType.DMA] * 2,
          [pltpu.SemaphoreType.DMA] * 2,
      ],
      compiler_params=pltpu.CompilerParams(use_tc_tiling_on_sc=False, needs_layout_passes=False),
  )(table, xt)


def kernel(x, table):
  xt = jnp.swapaxes(x, 0, 1).reshape(_SEQ * _S)
  o = _lookup(table, xt)
  o = o.reshape(_SEQ, 8, _S // 128, 8, 128)
  o = o.transpose(2, 4, 0, 1, 3)
  return o.reshape(_S, _SEQ, EMB)


# speedup vs baseline: 2.1698x; 2.1698x over previous
import functools
import math

import jax
import jax.numpy as jnp
from jax import lax
from jax.experimental import pallas as pl
from jax.experimental.pallas import tpu as pltpu
from jax.experimental.pallas import tpu_sc as plsc

EMB = 64
SCALE = math.sqrt(EMB)

_NC = 2
_NS = 16
_NW = _NC * _NS

_BLK = 256
_TP = 273
_SEQ = 200
_S = 16384
_SPW = _S // _NW


def _tile_body(table_hbm, x_hbm, out_hbm, idx_v, rows_v, tbuf_v, isem, gsem,
               wsem):
  wid = lax.axis_index("s") * _NC + lax.axis_index("c")
  n_blocks = _SEQ * (_SPW // _BLK)

  iota16 = lax.iota(jnp.int32, 16)

  def xoff(k):
    t = k // 2
    sub = k % 2
    return t * _S + wid * _SPW + sub * _BLK

  def start_idx(k, b):
    pltpu.async_copy(x_hbm.at[pl.ds(xoff(k), _BLK)], idx_v.at[b], isem[b])

  def wait_idx(b):
    pltpu.make_async_copy(x_hbm.at[pl.ds(0, _BLK)], idx_v.at[b],
                          isem[b]).wait()

  def start_gather(b):
    pltpu.async_copy(table_hbm.at[idx_v.at[b]], rows_v.at[b], gsem[b])

  def wait_gather(b):
    pltpu.make_async_copy(table_hbm.at[idx_v.at[b]], rows_v.at[b],
                          gsem[b]).wait()

  def start_write(k, b):
    t = k // 2
    sub = k % 2
    j0 = wid * 4 + sub * 2
    for j in range(2):
      pltpu.async_copy(tbuf_v.at[b, :, :, pl.ds(j * 128, 128)],
                       out_hbm.at[t, :, j0 + j, :, :], wsem[b])

  def wait_write(b):
    for j in range(2):
      pltpu.make_async_copy(tbuf_v.at[b, :, :, pl.ds(j * 128, 128)],
                            out_hbm.at[0, :, 0, :, :], wsem[b]).wait()

  q_ids = [(iota16 + g * 16) // 8 for g in range(4)]
  e8_ids = [(iota16 + g * 16) % 8 for g in range(4)]

  def transpose_scale(b):
    rows = rows_v.at[b]
    tb = tbuf_v.at[b]

    def row_body(r, carry):
      col = jnp.full((16,), r, jnp.int32)
      for g in range(4):
        vals = rows[r, pl.ds(g * 16, 16)] * SCALE
        plsc.store_scatter(tb, [q_ids[g], e8_ids[g], col], vals)
      return carry

    lax.fori_loop(0, _BLK, row_body, 0)

  def step(k, b, *, do_idx=True, do_gather=True, do_wwait=True):
    b1 = (b + 1) % 2
    wait_gather(b)
    if do_idx:
      start_idx(k + 2, b)
    if do_gather:
      wait_idx(b1)
      start_gather(b1)
    if do_wwait:
      wait_write(b)
    transpose_scale(b)
    start_write(k, b)

  start_idx(0, 0)
  start_idx(1, 1)
  wait_idx(0)
  start_gather(0)
  step(0, 0, do_wwait=False)
  step(1, 1, do_wwait=False)

  def pair(p, carry):
    step(2 * p, 0)
    step(2 * p + 1, 1)
    return carry

  lax.fori_loop(1, n_blocks // 2 - 1, pair, 0)

  step(n_blocks - 2, 0, do_idx=False)
  step(n_blocks - 1, 1, do_idx=False, do_gather=False)
  wait_write(0)
  wait_write(1)


@jax.jit
def _lookup(table, xt):
  mesh = plsc.VectorSubcoreMesh(core_axis_name="c", subcore_axis_name="s",
                                num_cores=_NC)
  return pl.kernel(
      _tile_body,
      out_type=jax.ShapeDtypeStruct((_SEQ, 8, _S // 128, 8, 128), jnp.float32),
      mesh=mesh,
      scratch_types=[
          pltpu.VMEM((2, _BLK), jnp.int32),
          pltpu.VMEM((2, _BLK, EMB), jnp.float32),
          pltpu.VMEM((2, 8, 8, _TP), jnp.float32),
          [pltpu.SemaphoreType.DMA] * 2,
          [pltpu.SemaphoreType.DMA] * 2,
          [pltpu.SemaphoreType.DMA] * 2,
      ],
      compiler_params=pltpu.CompilerParams(use_tc_tiling_on_sc=False, needs_layout_passes=False),
  )(table, xt)


def kernel(x, table):
  xt = jnp.swapaxes(x, 0, 1).reshape(_SEQ * _S)
  o = _lookup(table, xt)
  o = o.transpose(2, 4, 0, 1, 3)
  return o.reshape(_S, _SEQ, EMB)


# speedup vs baseline: 2.2135x; 1.0201x over previous
import functools
import math

import jax
import jax.numpy as jnp
from jax import lax
from jax.experimental import pallas as pl
from jax.experimental.pallas import tpu as pltpu
from jax.experimental.pallas import tpu_sc as plsc

EMB = 64
SCALE = math.sqrt(EMB)

_NC = 2
_NS = 16
_NW = _NC * _NS

_BLK = 256
_TP = 273
_SEQ = 200
_S = 16384
_SPW = _S // _NW


def _tile_body(table_hbm, x_hbm, out_hbm, idx_v, rows_v, tbuf_v, isem, gsem,
               wsem):
  wid = lax.axis_index("s") * _NC + lax.axis_index("c")
  n_blocks = _SEQ * (_SPW // _BLK)

  iota16 = lax.iota(jnp.int32, 16)

  def xoff(k):
    t = k // 2
    sub = k % 2
    return t * _S + wid * _SPW + sub * _BLK

  def start_idx(k, b):
    pltpu.async_copy(x_hbm.at[pl.ds(xoff(k), _BLK)], idx_v.at[b], isem[b])

  def wait_idx(b):
    pltpu.make_async_copy(x_hbm.at[pl.ds(0, _BLK)], idx_v.at[b],
                          isem[b]).wait()

  def start_gather(b):
    pltpu.async_copy(table_hbm.at[idx_v.at[b]], rows_v.at[b], gsem[b])

  def wait_gather(b):
    pltpu.make_async_copy(table_hbm.at[idx_v.at[b]], rows_v.at[b],
                          gsem[b]).wait()

  def start_write(k, b):
    t = k // 2
    sub = k % 2
    j0 = wid * 4 + sub * 2
    pltpu.async_copy(tbuf_v.at[b],
                     out_hbm.at[t, :, pl.ds(j0, 2), :, :], wsem[b])

  def wait_write(b):
    pltpu.make_async_copy(tbuf_v.at[b],
                          out_hbm.at[0, :, pl.ds(0, 2), :, :], wsem[b]).wait()

  e16s = [(iota16 + d) % 16 for d in range(16)]
  rcols = e16s
  wqs = [e // 8 for e in e16s]
  we8s = [e % 8 for e in e16s]

  def transpose_scale(b):
    rows = rows_v.at[b]
    tb = tbuf_v.at[b]

    def sub_body(cg, carry):
      c = cg // 4
      g = cg % 4
      j = c // 8
      rowv = iota16 + c * 16
      jv = jnp.full((16,), j, jnp.int32)
      sv = iota16 + (c % 8) * 16
      for d in range(16):
        vals = plsc.load_gather(rows, [rowv, rcols[d] + g * 16]) * SCALE
        plsc.store_scatter(tb, [wqs[d] + 2 * g, jv, we8s[d], sv], vals)
      return carry

    lax.fori_loop(0, 64, sub_body, 0)

  def step(k, b, *, do_idx=True, do_gather=True, do_wwait=True):
    b1 = (b + 1) % 2
    wait_gather(b)
    if do_idx:
      start_idx(k + 2, b)
    if do_gather:
      wait_idx(b1)
      start_gather(b1)
    if do_wwait:
      wait_write(b)
    transpose_scale(b)
    start_write(k, b)

  start_idx(0, 0)
  start_idx(1, 1)
  wait_idx(0)
  start_gather(0)
  step(0, 0, do_wwait=False)
  step(1, 1, do_wwait=False)

  def pair(p, carry):
    step(2 * p, 0)
    step(2 * p + 1, 1)
    return carry

  lax.fori_loop(1, n_blocks // 2 - 1, pair, 0)

  step(n_blocks - 2, 0, do_idx=False)
  step(n_blocks - 1, 1, do_idx=False, do_gather=False)
  wait_write(0)
  wait_write(1)


@jax.jit
def _lookup(table, xt):
  mesh = plsc.VectorSubcoreMesh(core_axis_name="c", subcore_axis_name="s",
                                num_cores=_NC)
  return pl.kernel(
      _tile_body,
      out_type=jax.ShapeDtypeStruct((_SEQ, 8, _S // 128, 8, 128), jnp.float32),
      mesh=mesh,
      scratch_types=[
          pltpu.VMEM((2, _BLK), jnp.int32),
          pltpu.VMEM((2, _BLK, EMB), jnp.float32),
          pltpu.VMEM((2, 8, 2, 8, 128), jnp.float32),
          [pltpu.SemaphoreType.DMA] * 2,
          [pltpu.SemaphoreType.DMA] * 2,
          [pltpu.SemaphoreType.DMA] * 2,
      ],
      compiler_params=pltpu.CompilerParams(use_tc_tiling_on_sc=False, needs_layout_passes=False),
  )(table, xt)


def kernel(x, table):
  xt = jnp.swapaxes(x, 0, 1).reshape(_SEQ * _S)
  o = _lookup(table, xt)
  o = o.transpose(2, 4, 0, 1, 3)
  return o.reshape(_S, _SEQ, EMB)


# speedup vs baseline: 4.4878x; 2.0275x over previous
import functools
import math

import jax
import jax.numpy as jnp
from jax import lax
from jax.experimental import pallas as pl
from jax.experimental.pallas import tpu as pltpu
from jax.experimental.pallas import tpu_sc as plsc

EMB = 64
SCALE = math.sqrt(EMB)

_NC = 2
_NS = 16
_NW = _NC * _NS

_BLK = 256
_TP = 273
_SEQ = 200
_S = 16384
_SPW = _S // _NW


def _tile_body(table_hbm, x_hbm, out_hbm, idx_v, rows_v, tbuf_v, isem, gsem,
               wsem):
  wid = lax.axis_index("s") * _NC + lax.axis_index("c")
  n_blocks = _SEQ * (_SPW // _BLK)

  iota16 = lax.iota(jnp.int32, 16)

  def xoff(k):
    t = k // 2
    sub = k % 2
    return t * _S + wid * _SPW + sub * _BLK

  def start_idx(k, b):
    pltpu.async_copy(x_hbm.at[pl.ds(xoff(k), _BLK)], idx_v.at[b], isem[b])

  def wait_idx(b):
    pltpu.make_async_copy(x_hbm.at[pl.ds(0, _BLK)], idx_v.at[b],
                          isem[b]).wait()

  def start_gather(b):
    pltpu.async_copy(table_hbm.at[idx_v.at[b]], rows_v.at[b], gsem[b])

  def wait_gather(b):
    pltpu.make_async_copy(table_hbm.at[idx_v.at[b]], rows_v.at[b],
                          gsem[b]).wait()

  def start_write(k, b):
    t = k // 2
    sub = k % 2
    j0 = wid * 4 + sub * 2
    pltpu.async_copy(tbuf_v.at[b],
                     out_hbm.at[t, :, pl.ds(j0, 2), :, :], wsem[b])

  def wait_write(b):
    pltpu.make_async_copy(tbuf_v.at[b],
                          out_hbm.at[0, :, pl.ds(0, 2), :, :], wsem[b]).wait()

  e16s = [(iota16 + d) % 16 for d in range(16)]
  rcols = e16s
  wqs = [e // 8 for e in e16s]
  we8s = [e % 8 for e in e16s]

  def transpose_scale(b):
    rows = rows_v.at[b]
    tb = tbuf_v.at[b]

    def sub_body(cg, carry):
      c = cg // 4
      g = cg % 4
      j = c // 8
      rowv = iota16 + c * 16
      jv = jnp.full((16,), j, jnp.int32)
      sv = iota16 + (c % 8) * 16
      vals = [plsc.load_gather(rows, [rowv, rcols[d] + g * 16])
              for d in range(16)]
      vals = [v * SCALE for v in vals]
      for d in range(16):
        plsc.store_scatter(tb, [wqs[d] + 2 * g, jv, we8s[d], sv], vals[d])
      return carry

    lax.fori_loop(0, 64, sub_body, 0)

  def step(k, b, *, do_idx=True, do_gather=True, do_wwait=True):
    b1 = (b + 1) % 2
    wait_gather(b)
    if do_idx:
      start_idx(k + 2, b)
    if do_gather:
      wait_idx(b1)
      start_gather(b1)
    if do_wwait:
      wait_write(b)
    transpose_scale(b)
    start_write(k, b)

  start_idx(0, 0)
  start_idx(1, 1)
  wait_idx(0)
  start_gather(0)
  step(0, 0, do_wwait=False)
  step(1, 1, do_wwait=False)

  def pair(p, carry):
    step(2 * p, 0)
    step(2 * p + 1, 1)
    return carry

  lax.fori_loop(1, n_blocks // 2 - 1, pair, 0)

  step(n_blocks - 2, 0, do_idx=False)
  step(n_blocks - 1, 1, do_idx=False, do_gather=False)
  wait_write(0)
  wait_write(1)


@jax.jit
def _lookup(table, xt):
  mesh = plsc.VectorSubcoreMesh(core_axis_name="c", subcore_axis_name="s",
                                num_cores=_NC)
  return pl.kernel(
      _tile_body,
      out_type=jax.ShapeDtypeStruct((_SEQ, 8, _S // 128, 8, 128), jnp.float32),
      mesh=mesh,
      scratch_types=[
          pltpu.VMEM((2, _BLK), jnp.int32),
          pltpu.VMEM((2, _BLK, EMB), jnp.float32),
          pltpu.VMEM((2, 8, 2, 8, 128), jnp.float32),
          [pltpu.SemaphoreType.DMA] * 2,
          [pltpu.SemaphoreType.DMA] * 2,
          [pltpu.SemaphoreType.DMA] * 2,
      ],
      compiler_params=pltpu.CompilerParams(use_tc_tiling_on_sc=False, needs_layout_passes=False),
  )(table, xt)


def kernel(x, table):
  xt = jnp.swapaxes(x, 0, 1).reshape(_SEQ * _S)
  o = _lookup(table, xt)
  o = o.transpose(2, 4, 0, 1, 3)
  return o.reshape(_S, _SEQ, EMB)
